# baseline (speedup 1.0000x reference)
import functools

import jax
import jax.numpy as jnp
from jax import lax
from jax.experimental import pallas as pl
from jax.experimental.pallas import tpu as pltpu
from jax.experimental.pallas import tpu_sc as plsc

N = 10000
D = 128
E = 320000

NC = 2
NS = 16
NW = NC * NS
EPW = E // NW
K = 80
NCHUNK = EPW // K
PADE = 2 * K
RT = 624
NTAIL = N - NS * RT
NVEC = D // 16


def _spmm_body(inp_hbm, val_hbm, row_hbm, col_hbm, out_hbm,
               colm, valm, rowx, gbuf, acc, gsem, rsem):
    c = lax.axis_index("c")
    s = lax.axis_index("s")
    w = c * NS + s

    def _wait_gather():
        pltpu.make_async_copy(inp_hbm.at[pl.ds(0, K)], gbuf.at[0], gsem).wait()

    def _wait_rowx():
        pltpu.make_async_copy(row_hbm.at[pl.ds(0, K)], rowx.at[0], rsem).wait()

    def _scale(j, half):
        for eb in range(K // 16):
            vvec = valm[pl.ds(j * K + eb * 16, 16)]
            for l in range(16):
                v16 = vvec.at[lax.broadcast(l, (16,))].get(
                    mode="promise_in_bounds")
                e = eb * 16 + l
                for q in range(NVEC):
                    gbuf[half, e, pl.ds(q * 16, 16)] = (
                        gbuf[half, e, pl.ds(q * 16, 16)] * v16)

    zeros16 = jnp.zeros((16,), jnp.float32)

    def _zero_row(i, carry):
        for q in range(NVEC):
            gbuf[0, i, pl.ds(q * 16, 16)] = zeros16
        return carry

    lax.fori_loop(0, K, _zero_row, None)
    r0 = s * RT
    for t in range(RT // K):
        pltpu.sync_copy(gbuf.at[0], acc.at[pl.ds(r0 + t * K, K)])
    rrem = RT - (RT // K) * K
    pltpu.sync_copy(gbuf.at[0, pl.ds(0, rrem)],
                    acc.at[pl.ds(r0 + (RT // K) * K, rrem)])

    @pl.when(s == 0)
    def _zero_tail():
        pltpu.sync_copy(gbuf.at[0, pl.ds(0, NTAIL)],
                        acc.at[pl.ds(NS * RT, NTAIL)])

    plsc.subcore_barrier()

    pltpu.sync_copy(col_hbm.at[pl.ds(w * EPW, EPW + PADE)], colm)
    pltpu.sync_copy(val_hbm.at[pl.ds(w * EPW, EPW)], valm)
    pltpu.async_copy(inp_hbm.at[colm.at[pl.ds(0, K)]], gbuf.at[0], gsem)
    pltpu.async_copy(inp_hbm.at[colm.at[pl.ds(K, K)]], gbuf.at[1], gsem)
    pltpu.async_copy(row_hbm.at[pl.ds(w * EPW, K)], rowx.at[0], rsem)
    pltpu.async_copy(row_hbm.at[pl.ds(w * EPW + K, K)], rowx.at[1], rsem)

    def _step(j, half):
        _wait_gather()
        _scale(j, half)
        _wait_rowx()
        pltpu.sync_copy(gbuf.at[half], acc.at[rowx.at[half]], add=True)
        pltpu.async_copy(inp_hbm.at[colm.at[pl.ds((j + 2) * K, K)]],
                         gbuf.at[half], gsem)
        pltpu.async_copy(row_hbm.at[pl.ds(w * EPW + (j + 2) * K, K)],
                         rowx.at[half], rsem)

    def _pair(p, carry):
        _step(p * 2, 0)
        _step(p * 2 + 1, 1)
        return carry

    lax.fori_loop(0, NCHUNK // 2, _pair, None)
    _step(NCHUNK - 1, 0)

    for _ in range(2):
        _wait_gather()
        _wait_rowx()
    plsc.subcore_barrier()

    for t in range(RT // K):
        pltpu.sync_copy(acc.at[pl.ds(r0 + t * K, K)], gbuf.at[0])
        pltpu.sync_copy(gbuf.at[0], out_hbm.at[c, pl.ds(r0 + t * K, K)])
    pltpu.sync_copy(acc.at[pl.ds(r0 + (RT // K) * K, rrem)],
                    gbuf.at[0, pl.ds(0, rrem)])
    pltpu.sync_copy(gbuf.at[0, pl.ds(0, rrem)],
                    out_hbm.at[c, pl.ds(r0 + (RT // K) * K, rrem)])

    @pl.when(s == 0)
    def _write_tail():
        pltpu.sync_copy(acc.at[pl.ds(NS * RT, NTAIL)],
                        gbuf.at[1, pl.ds(0, NTAIL)])
        pltpu.sync_copy(gbuf.at[1, pl.ds(0, NTAIL)],
                        out_hbm.at[c, pl.ds(NS * RT, NTAIL)])


_spmm_sc = functools.partial(
    pl.kernel,
    out_type=jax.ShapeDtypeStruct((NC, N, D), jnp.float32),
    mesh=plsc.VectorSubcoreMesh(core_axis_name="c", subcore_axis_name="s"),
    scratch_types=[
        pltpu.VMEM((EPW + PADE,), jnp.int32),
        pltpu.VMEM((EPW,), jnp.float32),
        pltpu.VMEM((2, K), jnp.int32),
        pltpu.VMEM((2, K, D), jnp.float32),
        pltpu.VMEM_SHARED((N, D), jnp.float32),
        pltpu.SemaphoreType.DMA,
        pltpu.SemaphoreType.DMA,
    ],
)(_spmm_body)


def _add_partials(p_ref, o_ref):
    o_ref[...] = p_ref[0] + p_ref[1]


def _sum_partials(partials):
    return pl.pallas_call(
        _add_partials,
        grid=(10,),
        in_specs=[pl.BlockSpec((2, N // 10, D), lambda i: (0, i, 0))],
        out_specs=pl.BlockSpec((N // 10, D), lambda i: (i, 0)),
        out_shape=jax.ShapeDtypeStruct((N, D), jnp.float32),
    )(partials)


def kernel(input, flow_char_adj_values, flow_char_adj_indices):
    idx = flow_char_adj_indices.astype(jnp.int32)
    zk = jnp.zeros((PADE,), jnp.int32)
    row = jnp.concatenate([idx[0], zk])
    col = jnp.concatenate([idx[1], zk])
    vals = flow_char_adj_values.astype(jnp.float32)
    partials = _spmm_sc(input, vals, row, col)
    return _sum_partials(partials)
